# baseline (speedup 1.0000x reference)
import functools

import jax
import jax.numpy as jnp
from jax.experimental import pallas as pl
from jax.experimental.pallas import tpu as pltpu
from jax.experimental.pallas import tpu_sc as plsc

N_EXPERTS = 8
TOP_K = 2
T = 2048
C = 768
HBLK = 512
RB = 128
S = T * TOP_K + N_EXPERTS * RB
RMAX = T * TOP_K
CSB = 128
CHUNK = 1024
WSCALE = 64.0
SC_W = 128


def _router_body(x_ref, wr_ref, pos0_ref, pos1_ref, w0_ref, w1_ref, sc_ref):
    x = x_ref[...]
    logits = jax.lax.dot_general(
        x, wr_ref[...], (((1,), (1,)), ((), ())),
        preferred_element_type=jnp.float32)
    m = jnp.max(logits, axis=-1, keepdims=True)
    p = jnp.exp(logits - m)
    gate = p / jnp.sum(p, axis=-1, keepdims=True)
    idx = jax.lax.broadcasted_iota(jnp.int32, gate.shape, 1)
    m1 = jnp.max(gate, axis=-1, keepdims=True)
    i1 = jnp.min(jnp.where(gate == m1, idx, N_EXPERTS), axis=-1, keepdims=True)
    oh1 = idx == i1
    g2 = jnp.where(oh1, -1.0, gate)
    m2 = jnp.max(g2, axis=-1, keepdims=True)
    i2 = jnp.min(jnp.where(g2 == m2, idx, N_EXPERTS), axis=-1, keepdims=True)
    oh2 = idx == i2
    w0_ref[...] = jnp.sum(jnp.where(oh1, gate, 0.0), axis=-1, keepdims=True)
    w1_ref[...] = jnp.sum(jnp.where(oh2, gate, 0.0), axis=-1, keepdims=True)

    r_iota = jax.lax.broadcasted_iota(jnp.int32, (CSB, CSB), 0)
    c_iota = jax.lax.broadcasted_iota(jnp.int32, (CSB, CSB), 1)
    ltri = jnp.where(c_iota < r_iota, 1.0, 0.0)
    carry = jnp.zeros((1, N_EXPERTS), jnp.float32)
    ranks = []
    for part in (oh1, oh2):
        mat = part.astype(jnp.float32)
        pieces = []
        for b in range(T // CSB):
            blk = mat[b * CSB:(b + 1) * CSB, :]
            pieces.append(jax.lax.dot_general(
                ltri, blk, (((1,), (0,)), ((), ())),
                preferred_element_type=jnp.float32) + carry)
            carry = carry + jnp.sum(blk, axis=0, keepdims=True)
        ranks.append(jnp.concatenate(pieces, axis=0))

    cnt = carry.astype(jnp.int32)
    nrb = (cnt + (RB - 1)) // RB
    padded_f = (nrb * RB).astype(jnp.float32)
    er = jax.lax.broadcasted_iota(jnp.int32, (N_EXPERTS, N_EXPERTS), 0)
    ec = jax.lax.broadcasted_iota(jnp.int32, (N_EXPERTS, N_EXPERTS), 1)
    utri = jnp.where(er < ec, 1.0, 0.0)
    offs_f = jax.lax.dot_general(padded_f, utri, (((1,), (0,)), ((), ())),
                                 preferred_element_type=jnp.float32)
    pos0 = jnp.sum(jnp.where(oh1, ranks[0] + offs_f, 0.0), axis=-1,
                   keepdims=True)
    pos1 = jnp.sum(jnp.where(oh2, ranks[1] + offs_f, 0.0), axis=-1,
                   keepdims=True)
    pos0_ref[...] = pos0.astype(jnp.int32)
    pos1_ref[...] = pos1.astype(jnp.int32)
    sc_ref[0:1, :] = offs_f.astype(jnp.int32)
    sc_ref[1:2, :] = nrb


def _router(x2d, w_router):
    return pl.pallas_call(
        _router_body,
        in_specs=[pl.BlockSpec((T, C), lambda: (0, 0)),
                  pl.BlockSpec((N_EXPERTS, C), lambda: (0, 0))],
        out_specs=[pl.BlockSpec((T, 1), lambda: (0, 0)),
                   pl.BlockSpec((T, 1), lambda: (0, 0)),
                   pl.BlockSpec((T, 1), lambda: (0, 0)),
                   pl.BlockSpec((T, 1), lambda: (0, 0)),
                   pl.BlockSpec((2, N_EXPERTS), lambda: (0, 0))],
        out_shape=[jax.ShapeDtypeStruct((T, 1), jnp.int32),
                   jax.ShapeDtypeStruct((T, 1), jnp.int32),
                   jax.ShapeDtypeStruct((T, 1), jnp.float32),
                   jax.ShapeDtypeStruct((T, 1), jnp.float32),
                   jax.ShapeDtypeStruct((2, N_EXPERTS), jnp.int32)],
    )(x2d, w_router)


def _sc_scatter(x2d, pos2):
    mesh = plsc.VectorSubcoreMesh(core_axis_name="c", subcore_axis_name="s")
    CH = C // 2

    @pl.kernel(out_type=jax.ShapeDtypeStruct((S, C), jnp.float32), mesh=mesh)
    def k(x_hbm, i_hbm, o_hbm):
        for ci in range(2):
            def body(x_vmem, i_vmem, ci=ci):
                pltpu.sync_copy(x_vmem,
                                o_hbm.at[i_vmem.at[0], pl.ds(ci * CH, CH)])

            pltpu.emit_pipeline(
                body,
                grid=(TOP_K, T // SC_W),
                in_specs=[
                    pl.BlockSpec((SC_W, CH),
                                 index_map=lambda k_, i, ci=ci: (i, ci)),
                    pl.BlockSpec((1, SC_W), index_map=lambda k_, i: (k_, i))],
                out_specs=[],
                core_axis_name=("c", "s"),
                dimension_semantics=(pltpu.PARALLEL, pltpu.PARALLEL),
            )(x_hbm, i_hbm)

    return k(x2d, pos2)


def _sc_gather(h_buf, pos_flat):
    mesh = plsc.VectorSubcoreMesh(core_axis_name="c", subcore_axis_name="s")
    nblk = (TOP_K * T) // SC_W
    CH = C // 2

    @pl.kernel(out_type=jax.ShapeDtypeStruct((TOP_K * T, C), jnp.float32),
               mesh=mesh)
    def k(h_hbm, i_hbm, o_hbm):
        for ci in range(2):
            def body(i_vmem, o_vmem, ci=ci):
                pltpu.sync_copy(h_hbm.at[i_vmem.at[0], pl.ds(ci * CH, CH)],
                                o_vmem)

            pltpu.emit_pipeline(
                body,
                grid=(2, nblk // 2),
                in_specs=[pl.BlockSpec(
                    (1, SC_W),
                    index_map=lambda k_, i: (0, k_ * (nblk // 2) + i))],
                out_specs=[pl.BlockSpec(
                    (SC_W, CH),
                    index_map=lambda k_, i, ci=ci: (k_ * (nblk // 2) + i, ci))],
                core_axis_name=("c", "s"),
                dimension_semantics=(pltpu.PARALLEL, pltpu.PARALLEL),
            )(i_hbm, o_hbm)

    return k(h_buf, pos_flat)


def _gemm_body(s_ref, x_hbm, g_ref, u_ref, d_ref, *rest, e, h_e, nh):
    if len(rest) == 7:
        _, hout_hbm, xs, xs8, acc, sem_i, sem_o = rest
    else:
        hout_hbm, xs, xs8, acc, sem_i, sem_o = rest
    j = pl.program_id(0)
    off = s_ref[0, e]
    n = s_ref[1, e]

    def _cp_in(ri):
        return pltpu.make_async_copy(
            x_hbm.at[pl.ds(pl.multiple_of(off + ri * RB, RB), RB), :],
            xs.at[pl.ds(pl.multiple_of(ri * RB, RB), RB), :], sem_i)

    def _cp_out(ri):
        return pltpu.make_async_copy(
            acc.at[pl.ds(pl.multiple_of(ri * RB, RB), RB), :],
            hout_hbm.at[pl.ds(pl.multiple_of(off + ri * RB, RB), RB), :],
            sem_o)

    @pl.when(j == 0)
    def _():
        jax.lax.fori_loop(0, n, lambda ri, c: (_cp_in(ri).start(), c)[1], 0)
        acc[...] = jnp.zeros_like(acc)
        jax.lax.fori_loop(0, n, lambda ri, c: (_cp_in(ri).wait(), c)[1], 0)
        xs8[...] = xs[...].astype(jnp.float8_e4m3fn)

    gb = g_ref[...]
    ub = u_ref[...]
    db = d_ref[...]
    tail = h_e % HBLK
    if tail:
        def _mask(o):
            gg, uu, dd = o
            row = jax.lax.broadcasted_iota(jnp.int32, gg.shape, 0)
            col = jax.lax.broadcasted_iota(jnp.int32, dd.shape, 1)
            return (jnp.where(row < tail, gg, 0.0),
                    jnp.where(row < tail, uu, 0.0),
                    jnp.where(col < tail, dd, 0.0))
        gb, ub, db = jax.lax.cond(j == nh - 1, _mask, lambda o: o,
                                  (gb, ub, db))
    g8w = (gb * WSCALE).astype(jnp.float8_e4m3fn)
    u8w = (ub * WSCALE).astype(jnp.float8_e4m3fn)
    d8w = (db * WSCALE).astype(jnp.float8_e4m3fn)

    for ci in range(RMAX // CHUNK):
        @pl.when(ci * (CHUNK // RB) < n)
        def _(ci=ci):
            sl = slice(ci * CHUNK, (ci + 1) * CHUNK)
            xr = xs8[sl, :]
            g = jax.lax.dot_general(xr, g8w, (((1,), (1,)), ((), ())),
                                    preferred_element_type=jnp.float32)
            u = jax.lax.dot_general(xr, u8w, (((1,), (1,)), ((), ())),
                                    preferred_element_type=jnp.float32)
            g = g * (1.0 / WSCALE)
            u = u * (1.0 / WSCALE)
            h = (g * jax.nn.sigmoid(g)) * u
            h8 = (h * WSCALE).astype(jnp.float8_e4m3fn)
            contrib = jax.lax.dot_general(h8, d8w, (((1,), (1,)), ((), ())),
                                          preferred_element_type=jnp.float32)
            acc[sl, :] += contrib * (1.0 / (WSCALE * WSCALE))

    @pl.when(j == nh - 1)
    def _():
        jax.lax.fori_loop(0, n, lambda ri, c: (_cp_out(ri).start(), c)[1], 0)
        jax.lax.fori_loop(0, n, lambda ri, c: (_cp_out(ri).wait(), c)[1], 0)


def _gemm_call(e, scalars, x_sorted, gate_w, up_w, down_w, h_in):
    h_e = gate_w.shape[0]
    nh = pl.cdiv(h_e, HBLK)
    in_specs = [
        pl.BlockSpec(memory_space=pl.ANY),
        pl.BlockSpec((HBLK, C), lambda j, s: (j, 0)),
        pl.BlockSpec((HBLK, C), lambda j, s: (j, 0)),
        pl.BlockSpec((C, HBLK), lambda j, s: (0, j)),
    ]
    args = [scalars, x_sorted, gate_w, up_w, down_w]
    aliases = {}
    if h_in is not None:
        in_specs.append(pl.BlockSpec(memory_space=pl.ANY))
        args.append(h_in)
        aliases = {5: 0}
    grid_spec = pltpu.PrefetchScalarGridSpec(
        num_scalar_prefetch=1,
        grid=(nh,),
        in_specs=in_specs,
        out_specs=pl.BlockSpec(memory_space=pl.ANY),
        scratch_shapes=[pltpu.VMEM((RMAX, C), jnp.float32),
                        pltpu.VMEM((RMAX, C), jnp.float8_e4m3fn),
                        pltpu.VMEM((RMAX, C), jnp.float32),
                        pltpu.SemaphoreType.DMA,
                        pltpu.SemaphoreType.DMA],
    )
    return pl.pallas_call(
        functools.partial(_gemm_body, e=e, h_e=h_e, nh=nh),
        grid_spec=grid_spec,
        out_shape=jax.ShapeDtypeStruct((S, C), jnp.float32),
        input_output_aliases=aliases,
        compiler_params=pltpu.CompilerParams(
            dimension_semantics=("arbitrary",)),
    )(*args)


def _combine_body(x_ref, g0_ref, g1_ref, w0_ref, w1_ref, o_ref):
    o_ref[...] = (x_ref[...] + w0_ref[...] * g0_ref[...]
                  + w1_ref[...] * g1_ref[...])


def _combine(x2d, g01, w0, w1):
    nb = 4
    rb = T // nb
    return pl.pallas_call(
        _combine_body,
        grid=(nb,),
        in_specs=[pl.BlockSpec((rb, C), lambda i: (i, 0)),
                  pl.BlockSpec((rb, C), lambda i: (i, 0)),
                  pl.BlockSpec((rb, C), lambda i: (i + nb, 0)),
                  pl.BlockSpec((rb, 1), lambda i: (i, 0)),
                  pl.BlockSpec((rb, 1), lambda i: (i, 0))],
        out_specs=pl.BlockSpec((rb, C), lambda i: (i, 0)),
        out_shape=jax.ShapeDtypeStruct((T, C), jnp.float32),
    )(x2d, g01, g01, w0, w1)


def kernel(x, w_router, gates, ups, downs):
    Bp, Tp, Cp = x.shape
    x2d = x.reshape(Tp, Cp)
    pos0, pos1, w0, w1, scalars = _router(x2d, w_router)
    pos2 = jnp.concatenate([pos0.reshape(1, T), pos1.reshape(1, T)], axis=0)
    x_sorted = _sc_scatter(x2d, pos2)
    h_buf = None
    for e in range(N_EXPERTS):
        h_buf = _gemm_call(e, scalars, x_sorted, gates[e], ups[e], downs[e],
                           h_buf)
    g01 = _sc_gather(h_buf, pos2.reshape(1, TOP_K * T))
    out = _combine(x2d, g01, w0, w1)
    return out.reshape(Bp, Tp, Cp)


# speedup vs baseline: 1.0461x; 1.0461x over previous
import functools

import jax
import jax.numpy as jnp
from jax.experimental import pallas as pl
from jax.experimental.pallas import tpu as pltpu
from jax.experimental.pallas import tpu_sc as plsc

N_EXPERTS = 8
TOP_K = 2
T = 2048
C = 768
HBLK = 512
RB = 128
S = T * TOP_K + N_EXPERTS * RB
RMAX = T * TOP_K
CSB = 128
CHUNK = 512
WSCALE = 64.0
SC_W = 128


def _router_body(x_ref, wr_ref, pos0_ref, pos1_ref, w0_ref, w1_ref, sc_ref):
    x = x_ref[...]
    logits = jax.lax.dot_general(
        x, wr_ref[...], (((1,), (1,)), ((), ())),
        preferred_element_type=jnp.float32)
    m = jnp.max(logits, axis=-1, keepdims=True)
    p = jnp.exp(logits - m)
    gate = p / jnp.sum(p, axis=-1, keepdims=True)
    idx = jax.lax.broadcasted_iota(jnp.int32, gate.shape, 1)
    m1 = jnp.max(gate, axis=-1, keepdims=True)
    i1 = jnp.min(jnp.where(gate == m1, idx, N_EXPERTS), axis=-1, keepdims=True)
    oh1 = idx == i1
    g2 = jnp.where(oh1, -1.0, gate)
    m2 = jnp.max(g2, axis=-1, keepdims=True)
    i2 = jnp.min(jnp.where(g2 == m2, idx, N_EXPERTS), axis=-1, keepdims=True)
    oh2 = idx == i2
    w0_ref[...] = jnp.sum(jnp.where(oh1, gate, 0.0), axis=-1, keepdims=True)
    w1_ref[...] = jnp.sum(jnp.where(oh2, gate, 0.0), axis=-1, keepdims=True)

    r_iota = jax.lax.broadcasted_iota(jnp.int32, (CSB, CSB), 0)
    c_iota = jax.lax.broadcasted_iota(jnp.int32, (CSB, CSB), 1)
    ltri = jnp.where(c_iota < r_iota, 1.0, 0.0)
    carry = jnp.zeros((1, N_EXPERTS), jnp.float32)
    ranks = []
    for part in (oh1, oh2):
        mat = part.astype(jnp.float32)
        pieces = []
        for b in range(T // CSB):
            blk = mat[b * CSB:(b + 1) * CSB, :]
            pieces.append(jax.lax.dot_general(
                ltri, blk, (((1,), (0,)), ((), ())),
                preferred_element_type=jnp.float32) + carry)
            carry = carry + jnp.sum(blk, axis=0, keepdims=True)
        ranks.append(jnp.concatenate(pieces, axis=0))

    cnt = carry.astype(jnp.int32)
    nrb = (cnt + (RB - 1)) // RB
    padded_f = (nrb * RB).astype(jnp.float32)
    er = jax.lax.broadcasted_iota(jnp.int32, (N_EXPERTS, N_EXPERTS), 0)
    ec = jax.lax.broadcasted_iota(jnp.int32, (N_EXPERTS, N_EXPERTS), 1)
    utri = jnp.where(er < ec, 1.0, 0.0)
    offs_f = jax.lax.dot_general(padded_f, utri, (((1,), (0,)), ((), ())),
                                 preferred_element_type=jnp.float32)
    pos0 = jnp.sum(jnp.where(oh1, ranks[0] + offs_f, 0.0), axis=-1,
                   keepdims=True)
    pos1 = jnp.sum(jnp.where(oh2, ranks[1] + offs_f, 0.0), axis=-1,
                   keepdims=True)
    pos0_ref[...] = pos0.astype(jnp.int32)
    pos1_ref[...] = pos1.astype(jnp.int32)
    sc_ref[0:1, :] = offs_f.astype(jnp.int32)
    sc_ref[1:2, :] = nrb


def _router(x2d, w_router):
    return pl.pallas_call(
        _router_body,
        in_specs=[pl.BlockSpec((T, C), lambda: (0, 0)),
                  pl.BlockSpec((N_EXPERTS, C), lambda: (0, 0))],
        out_specs=[pl.BlockSpec((T, 1), lambda: (0, 0)),
                   pl.BlockSpec((T, 1), lambda: (0, 0)),
                   pl.BlockSpec((T, 1), lambda: (0, 0)),
                   pl.BlockSpec((T, 1), lambda: (0, 0)),
                   pl.BlockSpec((2, N_EXPERTS), lambda: (0, 0))],
        out_shape=[jax.ShapeDtypeStruct((T, 1), jnp.int32),
                   jax.ShapeDtypeStruct((T, 1), jnp.int32),
                   jax.ShapeDtypeStruct((T, 1), jnp.float32),
                   jax.ShapeDtypeStruct((T, 1), jnp.float32),
                   jax.ShapeDtypeStruct((2, N_EXPERTS), jnp.int32)],
    )(x2d, w_router)


def _sc_scatter(x2d, pos2):
    mesh = plsc.VectorSubcoreMesh(core_axis_name="c", subcore_axis_name="s")
    CH = C // 2

    @pl.kernel(out_type=jax.ShapeDtypeStruct((S, C), jnp.float32), mesh=mesh)
    def k(x_hbm, i_hbm, o_hbm):
        for ci in range(2):
            def body(x_vmem, i_vmem, ci=ci):
                pltpu.sync_copy(x_vmem,
                                o_hbm.at[i_vmem.at[0], pl.ds(ci * CH, CH)])

            pltpu.emit_pipeline(
                body,
                grid=(TOP_K, T // SC_W),
                in_specs=[
                    pl.BlockSpec((SC_W, CH),
                                 index_map=lambda k_, i, ci=ci: (i, ci)),
                    pl.BlockSpec((1, SC_W), index_map=lambda k_, i: (k_, i))],
                out_specs=[],
                core_axis_name=("c", "s"),
                dimension_semantics=(pltpu.PARALLEL, pltpu.PARALLEL),
            )(x_hbm, i_hbm)

    return k(x2d, pos2)


def _sc_gather(h_buf, pos_flat):
    mesh = plsc.VectorSubcoreMesh(core_axis_name="c", subcore_axis_name="s")
    nblk = (TOP_K * T) // SC_W
    CH = C // 2

    @pl.kernel(out_type=jax.ShapeDtypeStruct((TOP_K * T, C), jnp.float32),
               mesh=mesh)
    def k(h_hbm, i_hbm, o_hbm):
        for ci in range(2):
            def body(i_vmem, o_vmem, ci=ci):
                pltpu.sync_copy(h_hbm.at[i_vmem.at[0], pl.ds(ci * CH, CH)],
                                o_vmem)

            pltpu.emit_pipeline(
                body,
                grid=(2, nblk // 2),
                in_specs=[pl.BlockSpec(
                    (1, SC_W),
                    index_map=lambda k_, i: (0, k_ * (nblk // 2) + i))],
                out_specs=[pl.BlockSpec(
                    (SC_W, CH),
                    index_map=lambda k_, i, ci=ci: (k_ * (nblk // 2) + i, ci))],
                core_axis_name=("c", "s"),
                dimension_semantics=(pltpu.PARALLEL, pltpu.PARALLEL),
            )(i_hbm, o_hbm)

    return k(h_buf, pos_flat)


def _gemm_body(s_ref, x_hbm, g_ref, u_ref, d_ref, *rest, e, h_e, nh):
    if len(rest) == 7:
        _, hout_hbm, xs, xs8, acc, sem_i, sem_o = rest
    else:
        hout_hbm, xs, xs8, acc, sem_i, sem_o = rest
    j = pl.program_id(0)
    off = s_ref[0, e]
    n = s_ref[1, e]

    def _cp_in(ri):
        return pltpu.make_async_copy(
            x_hbm.at[pl.ds(pl.multiple_of(off + ri * RB, RB), RB), :],
            xs.at[pl.ds(pl.multiple_of(ri * RB, RB), RB), :], sem_i)

    def _cp_out(ri):
        return pltpu.make_async_copy(
            acc.at[pl.ds(pl.multiple_of(ri * RB, RB), RB), :],
            hout_hbm.at[pl.ds(pl.multiple_of(off + ri * RB, RB), RB), :],
            sem_o)

    @pl.when(j == 0)
    def _():
        jax.lax.fori_loop(0, n, lambda ri, c: (_cp_in(ri).start(), c)[1], 0)
        acc[...] = jnp.zeros_like(acc)
        jax.lax.fori_loop(0, n, lambda ri, c: (_cp_in(ri).wait(), c)[1], 0)
        xs8[...] = xs[...].astype(jnp.float8_e4m3fn)

    gb = g_ref[...]
    ub = u_ref[...]
    db = d_ref[...]
    tail = h_e % HBLK
    if tail:
        def _mask(o):
            gg, uu, dd = o
            row = jax.lax.broadcasted_iota(jnp.int32, gg.shape, 0)
            col = jax.lax.broadcasted_iota(jnp.int32, dd.shape, 1)
            return (jnp.where(row < tail, gg, 0.0),
                    jnp.where(row < tail, uu, 0.0),
                    jnp.where(col < tail, dd, 0.0))
        gb, ub, db = jax.lax.cond(j == nh - 1, _mask, lambda o: o,
                                  (gb, ub, db))
    g8w = (gb * WSCALE).astype(jnp.float8_e4m3fn)
    u8w = (ub * WSCALE).astype(jnp.float8_e4m3fn)
    d8w = (db * WSCALE).astype(jnp.float8_e4m3fn)

    for ci in range(RMAX // CHUNK):
        @pl.when(ci * (CHUNK // RB) < n)
        def _(ci=ci):
            sl = slice(ci * CHUNK, (ci + 1) * CHUNK)
            xr = xs8[sl, :]
            g = jax.lax.dot_general(xr, g8w, (((1,), (1,)), ((), ())),
                                    preferred_element_type=jnp.float32)
            u = jax.lax.dot_general(xr, u8w, (((1,), (1,)), ((), ())),
                                    preferred_element_type=jnp.float32)
            g = g * (1.0 / WSCALE)
            u = u * (1.0 / WSCALE)
            h = (g * jax.nn.sigmoid(g)) * u
            h8 = (h * WSCALE).astype(jnp.float8_e4m3fn)
            contrib = jax.lax.dot_general(h8, d8w, (((1,), (1,)), ((), ())),
                                          preferred_element_type=jnp.float32)
            acc[sl, :] += contrib * (1.0 / (WSCALE * WSCALE))

    @pl.when(j == nh - 1)
    def _():
        jax.lax.fori_loop(0, n, lambda ri, c: (_cp_out(ri).start(), c)[1], 0)
        jax.lax.fori_loop(0, n, lambda ri, c: (_cp_out(ri).wait(), c)[1], 0)


def _gemm_call(e, scalars, x_sorted, gate_w, up_w, down_w, h_in):
    h_e = gate_w.shape[0]
    nh = pl.cdiv(h_e, HBLK)
    in_specs = [
        pl.BlockSpec(memory_space=pl.ANY),
        pl.BlockSpec((HBLK, C), lambda j, s: (j, 0)),
        pl.BlockSpec((HBLK, C), lambda j, s: (j, 0)),
        pl.BlockSpec((C, HBLK), lambda j, s: (0, j)),
    ]
    args = [scalars, x_sorted, gate_w, up_w, down_w]
    aliases = {}
    if h_in is not None:
        in_specs.append(pl.BlockSpec(memory_space=pl.ANY))
        args.append(h_in)
        aliases = {5: 0}
    grid_spec = pltpu.PrefetchScalarGridSpec(
        num_scalar_prefetch=1,
        grid=(nh,),
        in_specs=in_specs,
        out_specs=pl.BlockSpec(memory_space=pl.ANY),
        scratch_shapes=[pltpu.VMEM((RMAX, C), jnp.float32),
                        pltpu.VMEM((RMAX, C), jnp.float8_e4m3fn),
                        pltpu.VMEM((RMAX, C), jnp.float32),
                        pltpu.SemaphoreType.DMA,
                        pltpu.SemaphoreType.DMA],
    )
    return pl.pallas_call(
        functools.partial(_gemm_body, e=e, h_e=h_e, nh=nh),
        grid_spec=grid_spec,
        out_shape=jax.ShapeDtypeStruct((S, C), jnp.float32),
        input_output_aliases=aliases,
        compiler_params=pltpu.CompilerParams(
            dimension_semantics=("arbitrary",)),
    )(*args)


def _combine_body(x_ref, g0_ref, g1_ref, w0_ref, w1_ref, o_ref):
    o_ref[...] = (x_ref[...] + w0_ref[...] * g0_ref[...]
                  + w1_ref[...] * g1_ref[...])


def _combine(x2d, g01, w0, w1):
    nb = 4
    rb = T // nb
    return pl.pallas_call(
        _combine_body,
        grid=(nb,),
        in_specs=[pl.BlockSpec((rb, C), lambda i: (i, 0)),
                  pl.BlockSpec((rb, C), lambda i: (i, 0)),
                  pl.BlockSpec((rb, C), lambda i: (i + nb, 0)),
                  pl.BlockSpec((rb, 1), lambda i: (i, 0)),
                  pl.BlockSpec((rb, 1), lambda i: (i, 0))],
        out_specs=pl.BlockSpec((rb, C), lambda i: (i, 0)),
        out_shape=jax.ShapeDtypeStruct((T, C), jnp.float32),
    )(x2d, g01, g01, w0, w1)


def kernel(x, w_router, gates, ups, downs):
    Bp, Tp, Cp = x.shape
    x2d = x.reshape(Tp, Cp)
    pos0, pos1, w0, w1, scalars = _router(x2d, w_router)
    pos2 = jnp.concatenate([pos0.reshape(1, T), pos1.reshape(1, T)], axis=0)
    x_sorted = _sc_scatter(x2d, pos2)
    h_buf = None
    for e in range(N_EXPERTS):
        h_buf = _gemm_call(e, scalars, x_sorted, gates[e], ups[e], downs[e],
                           h_buf)
    g01 = _sc_gather(h_buf, pos2.reshape(1, TOP_K * T))
    out = _combine(x2d, g01, w0, w1)
    return out.reshape(Bp, Tp, Cp)
